# baseline (speedup 1.0000x reference)
import functools

import jax
import jax.numpy as jnp
from jax.experimental import pallas as pl
from jax.experimental.pallas import tpu as pltpu

_BN = 1024
_BK = 2048


def _dcn_loss_kernel(emb_ref, cent_ref, out_ref, acc_ref, *, inv_n):
    i = pl.program_id(0)
    j = pl.program_id(1)
    nj = pl.num_programs(1)

    x = emb_ref[...]
    ct = cent_ref[...]

    c_sq = jnp.sum(ct * ct, axis=0, keepdims=True)
    xc = jnp.dot(x, ct, preferred_element_type=jnp.float32)
    part = jnp.min(c_sq - 2.0 * xc, axis=1, keepdims=True)

    @pl.when(j == 0)
    def _init():
        acc_ref[...] = part

    @pl.when(j != 0)
    def _fold():
        acc_ref[...] = jnp.minimum(acc_ref[...], part)

    @pl.when(j == nj - 1)
    def _finish():
        x_sq = jnp.sum(x * x, axis=1, keepdims=True)
        s = jnp.sum(acc_ref[...] + x_sq) * inv_n

        @pl.when(i == 0)
        def _first():
            out_ref[0, 0] = s

        @pl.when(i != 0)
        def _rest():
            out_ref[0, 0] = out_ref[0, 0] + s


def kernel(embedded, centers):
    n, d = embedded.shape
    k, _ = centers.shape
    ni, nj = n // _BN, k // _BK

    total = pl.pallas_call(
        functools.partial(_dcn_loss_kernel, inv_n=1.0 / n),
        grid=(ni, nj),
        in_specs=[
            pl.BlockSpec((_BN, d), lambda i, j: (i, 0)),
            pl.BlockSpec((d, _BK), lambda i, j: (0, j)),
        ],
        out_specs=pl.BlockSpec(memory_space=pltpu.SMEM),
        out_shape=jax.ShapeDtypeStruct((1, 1), jnp.float32),
        scratch_shapes=[pltpu.VMEM((_BN, 1), jnp.float32)],
        compiler_params=pltpu.CompilerParams(
            dimension_semantics=("arbitrary", "arbitrary")
        ),
    )(embedded, centers.T)
    return total[0, 0]
